# baseline (speedup 1.0000x reference)
import functools

import jax
import jax.numpy as jnp
from jax import lax
from jax.experimental import pallas as pl
from jax.experimental.pallas import tpu as pltpu
from jax.experimental.pallas import tpu_sc as plsc

_VOCAB = 1_000_000
_DIM = 64
_BATCH = 16384
_NEGS = 15
_COLS = _NEGS + 1

_NC = 2
_NS = 16
_NW = _NC * _NS
_L = 16
_BPW = _BATCH // _NW
_CHUNK = 64
_NCHUNK = _BPW // _CHUNK


def _neg_indices():
    neg = jax.random.randint(jax.random.key(12345), (_BATCH, _NEGS), 0, _VOCAB)
    return jnp.transpose(neg.astype(jnp.int32)).reshape(-1)


def _vperm(v, idx):
    return lax.gather(
        v, idx.reshape(_L, 1),
        dimension_numbers=lax.GatherDimensionNumbers(
            offset_dims=(), collapsed_slice_dims=(0,), start_index_map=(0,)),
        slice_sizes=(1,),
        mode=lax.GatherScatterMode.PROMISE_IN_BOUNDS)


def _transpose_sum(partials):
    lane = lax.iota(jnp.int32, _L)
    vecs = list(partials)
    for k in range(4):
        b = 1 << k
        lo_mask = ((lane >> k) & 1) == 0
        idx_dn = (lane - b) & (_L - 1)
        idx_up = (lane + b) & (_L - 1)
        new = [None] * _L
        for i in range(_L):
            if (i >> k) & 1:
                continue
            a, bb = vecs[i], vecs[i + b]
            new[i] = jnp.where(lo_mask, a, _vperm(bb, idx_dn))
            new[i + b] = jnp.where(lo_mask, _vperm(a, idx_up), bb)
        vecs = new
    while len(vecs) > 1:
        vecs = [vecs[i] + vecs[i + 1] for i in range(0, len(vecs), 2)]
    return vecs[0]


def _sc_dots(target, context, neg, W_target, W_context):
    mesh = plsc.VectorSubcoreMesh(core_axis_name="c", subcore_axis_name="s")

    @functools.partial(
        pl.kernel,
        mesh=mesh,
        compiler_params=pltpu.CompilerParams(use_tc_tiling_on_sc=False),
        out_type=jax.ShapeDtypeStruct((_BATCH * _COLS,), jnp.float32),
        scratch_types=(
            [pltpu.VMEM((_CHUNK,), jnp.int32),
             pltpu.VMEM((_CHUNK, _DIM), jnp.float32)]
            + [pltpu.VMEM((_CHUNK, _DIM), jnp.float32) for _ in range(_COLS)]
            + [pltpu.VMEM((_CHUNK * _COLS,), jnp.float32),
               pltpu.SemaphoreType.DMA]
        ),
    )
    def k(target_hbm, context_hbm, neg_hbm, wt_hbm, wc_hbm, out_hbm,
          idx_v, emb_t, *rest):
        bufs = list(rest[:_COLS])
        dots_v = rest[_COLS]
        sem = rest[_COLS + 1]
        wid = lax.axis_index("s") * _NC + lax.axis_index("c")
        base = wid * _BPW
        lane = lax.iota(jnp.int32, _L)
        sign = jnp.where(lane == 0, 1.0, -1.0)

        def cbody(cb, carry):
            cbase = base + cb * _CHUNK
            pltpu.sync_copy(target_hbm.at[pl.ds(cbase, _CHUNK)], idx_v)
            pltpu.async_copy(wt_hbm.at[idx_v], emb_t, sem).wait()
            pltpu.sync_copy(context_hbm.at[pl.ds(cbase, _CHUNK)], idx_v)
            pltpu.async_copy(wc_hbm.at[idx_v], bufs[0], sem).wait()
            for j in range(_NEGS):
                pltpu.sync_copy(neg_hbm.at[pl.ds(j * _BATCH + cbase, _CHUNK)],
                                idx_v)
                pltpu.async_copy(wc_hbm.at[idx_v], bufs[j + 1], sem).wait()

            def ebody(i, ec):
                t = [emb_t[i, pl.ds(kk * _L, _L)] for kk in range(_DIM // _L)]
                partials = []
                for j in range(_COLS):
                    x0 = bufs[j][i, pl.ds(0, _L)] * t[0]
                    x1 = bufs[j][i, pl.ds(_L, _L)] * t[1]
                    x2 = bufs[j][i, pl.ds(2 * _L, _L)] * t[2]
                    x3 = bufs[j][i, pl.ds(3 * _L, _L)] * t[3]
                    partials.append((x0 + x1) + (x2 + x3))
                dots_row = _transpose_sum(partials) * sign
                dots_v[pl.ds(i * _COLS, _COLS)] = dots_row
                return ec
            lax.fori_loop(0, _CHUNK, ebody, 0)

            pltpu.sync_copy(dots_v,
                            out_hbm.at[pl.ds(cbase * _COLS, _CHUNK * _COLS)])
            return carry
        lax.fori_loop(0, _NCHUNK, cbody, 0)

    return k(target, context, neg, W_target, W_context)


def _tc_loss_body(dots_ref, out_ref):
    x = dots_ref[...]
    ls = jnp.minimum(x, 0.0) - jnp.log(1.0 + jnp.exp(-jnp.abs(x)))
    out_ref[...] = jnp.broadcast_to(-jnp.sum(ls) / _BATCH, (1, 1))


_tc_loss = pl.pallas_call(
    _tc_loss_body,
    out_shape=jax.ShapeDtypeStruct((1, 1), jnp.float32),
)


def kernel(target, context, W_target, W_context):
    neg = _neg_indices()
    dots = _sc_dots(target.astype(jnp.int32), context.astype(jnp.int32),
                    neg, W_target, W_context)
    loss = _tc_loss(dots.reshape(_BATCH * _COLS // 128, 128))
    return loss[0, 0]


# speedup vs baseline: 1.0374x; 1.0374x over previous
import functools

import jax
import jax.numpy as jnp
from jax import lax
from jax.experimental import pallas as pl
from jax.experimental.pallas import tpu as pltpu
from jax.experimental.pallas import tpu_sc as plsc

_VOCAB = 1_000_000
_DIM = 64
_BATCH = 16384
_NEGS = 15
_COLS = _NEGS + 1

_NC = 2
_NS = 16
_NW = _NC * _NS
_L = 16
_BPW = _BATCH // _NW
_CHUNK = 32
_NCHUNK = _BPW // _CHUNK
_TROW = 8
_NTILE = _VOCAB // _TROW
_TBLK = 32
_TPW = 3906
_NBLK = -(-(_NTILE - 31 * _TPW) // _TBLK)


def _neg_indices():
    neg = jax.random.randint(jax.random.key(12345), (_BATCH, _NEGS), 0, _VOCAB)
    return jnp.transpose(neg.astype(jnp.int32)).reshape(-1)


def _vperm(v, idx):
    return lax.gather(
        v, idx.reshape(_L, 1),
        dimension_numbers=lax.GatherDimensionNumbers(
            offset_dims=(), collapsed_slice_dims=(0,), start_index_map=(0,)),
        slice_sizes=(1,),
        mode=lax.GatherScatterMode.PROMISE_IN_BOUNDS)


def _transpose_sum(partials):
    lane = lax.iota(jnp.int32, _L)
    vecs = list(partials)
    for k in range(4):
        b = 1 << k
        lo_mask = ((lane >> k) & 1) == 0
        idx_dn = (lane - b) & (_L - 1)
        idx_up = (lane + b) & (_L - 1)
        new = [None] * _L
        for i in range(_L):
            if (i >> k) & 1:
                continue
            a, bb = vecs[i], vecs[i + b]
            new[i] = jnp.where(lo_mask, a, _vperm(bb, idx_dn))
            new[i + b] = jnp.where(lo_mask, _vperm(a, idx_up), bb)
        vecs = new
    while len(vecs) > 1:
        vecs = [vecs[i] + vecs[i + 1] for i in range(0, len(vecs), 2)]
    return vecs[0]


def _sc_compact(W3):
    mesh = plsc.VectorSubcoreMesh(core_axis_name="c", subcore_axis_name="s")

    @functools.partial(
        pl.kernel,
        mesh=mesh,
        out_type=jax.ShapeDtypeStruct((_VOCAB // 2, 2 * _DIM), jnp.float32),
        scratch_types=[
            pltpu.VMEM((_TBLK, _TROW, _DIM), jnp.float32),
            pltpu.VMEM((_TBLK, _TROW, _DIM), jnp.float32),
            pltpu.VMEM((_TBLK * _TROW // 2, 2 * _DIM), jnp.float32),
            pltpu.SemaphoreType.DMA,
            pltpu.SemaphoreType.DMA,
        ],
    )
    def k(w3_hbm, out_hbm, buf0, buf1, packed, sem0, sem1):
        wid = lax.axis_index("s") * _NC + lax.axis_index("c")
        wstart = wid * _TPW
        bufs = (buf0, buf1)
        sems = (sem0, sem1)

        def tstart_of(b):
            return jnp.minimum(wstart + b * _TBLK, _NTILE - _TBLK)

        h0 = pltpu.async_copy(w3_hbm.at[pl.ds(tstart_of(0), _TBLK)], buf0, sem0)
        h0.wait()

        def pbody(p, carry):
            for parity in range(2):
                b = p * 2 + parity
                cur = bufs[parity]
                nxt = bufs[1 - parity]
                nh = pltpu.async_copy(
                    w3_hbm.at[pl.ds(tstart_of(b + 1), _TBLK)], nxt,
                    sems[1 - parity])

                def gbody(g, gc):
                    for r in range(_TROW):
                        for kk in range(_DIM // _L):
                            packed[g * (_TROW // 2) + r // 2,
                                   pl.ds((r % 2) * _DIM + kk * _L, _L)] = (
                                cur[g, r, pl.ds(kk * _L, _L)])
                    return gc
                lax.fori_loop(0, _TBLK, gbody, 0)
                pltpu.sync_copy(
                    packed,
                    out_hbm.at[pl.ds(tstart_of(b) * (_TROW // 2),
                                     _TBLK * _TROW // 2), :])
                nh.wait()
            return carry
        lax.fori_loop(0, _NBLK // 2, pbody, 0)
        if _NBLK % 2 == 1:
            b = _NBLK - 1
            def gtail(g, gc):
                for r in range(_TROW):
                    for kk in range(_DIM // _L):
                        packed[g * (_TROW // 2) + r // 2,
                               pl.ds((r % 2) * _DIM + kk * _L, _L)] = (
                            buf0[g, r, pl.ds(kk * _L, _L)])
                return gc
            lax.fori_loop(0, _TBLK, gtail, 0)
            pltpu.sync_copy(
                packed,
                out_hbm.at[pl.ds(tstart_of(b) * (_TROW // 2),
                                 _TBLK * _TROW // 2), :])
            pltpu.async_copy(
                w3_hbm.at[pl.ds(tstart_of(b + 1), _TBLK)], buf1, sem1).wait()

    return k(W3)


def _sc_dots(target, context, neg, w3t, wlin):
    mesh = plsc.VectorSubcoreMesh(core_axis_name="c", subcore_axis_name="s")

    @functools.partial(
        pl.kernel,
        mesh=mesh,
        out_type=jax.ShapeDtypeStruct((_BATCH * _COLS,), jnp.float32),
        scratch_types=[
            pltpu.VMEM((_CHUNK,), jnp.int32),
            pltpu.VMEM((_CHUNK,), jnp.int32),
            pltpu.VMEM((_CHUNK,), jnp.int32),
            pltpu.VMEM((_CHUNK,), jnp.int32),
            pltpu.VMEM((_CHUNK,), jnp.int32),
            pltpu.VMEM((_CHUNK, _DIM), jnp.float32),
            pltpu.VMEM((_CHUNK, 2 * _DIM), jnp.float32),
            pltpu.VMEM((_CHUNK, 2 * _DIM), jnp.float32),
            pltpu.VMEM((_CHUNK * _COLS * _L,), jnp.float32),
            pltpu.VMEM((_CHUNK * _COLS,), jnp.float32),
            pltpu.SemaphoreType.DMA,
            pltpu.SemaphoreType.DMA,
            pltpu.SemaphoreType.DMA,
        ],
    )
    def k(target_hbm, context_hbm, neg_hbm, wt_hbm, wlin_hbm, out_hbm,
          xid0, xid1, pid0, pid1, tid_v, emb_t, xb0, xb1, part_v, dots_v,
          sem0, sem1, semt):
        wid = lax.axis_index("s") * _NC + lax.axis_index("c")
        base = wid * _BPW
        lane = lax.iota(jnp.int32, _L)
        sign = jnp.where(lane == 0, 1.0, -1.0)
        xids = (xid0, xid1)
        pids = (pid0, pid1)
        xbs = (xb0, xb1)
        sems = (sem0, sem1)

        def load_xidx(j, cbase, slot):
            if j == 0:
                pltpu.sync_copy(context_hbm.at[pl.ds(cbase, _CHUNK)],
                                xids[slot])
            else:
                pltpu.sync_copy(
                    neg_hbm.at[pl.ds((j - 1) * _BATCH + cbase, _CHUNK)],
                    xids[slot])

            def qbody(g, carry):
                v = xids[slot][pl.ds(g * _L, _L)]
                pids[slot][pl.ds(g * _L, _L)] = lax.shift_right_logical(v, 1)
                return carry
            lax.fori_loop(0, _CHUNK // _L, qbody, 0)
            return pltpu.async_copy(wlin_hbm.at[pids[slot]], xbs[slot],
                                    sems[slot])

        def cbody(cb, carry):
            cbase = base + cb * _CHUNK
            pltpu.sync_copy(target_hbm.at[pl.ds(cbase, _CHUNK)], tid_v)
            for g in range(_CHUNK // _L):
                vt = tid_v[pl.ds(g * _L, _L)]
                handles = []
                for l in range(_L):
                    i = g * _L + l
                    r = vt[l]
                    handles.append(pltpu.async_copy(
                        wt_hbm.at[r >> 3, r & (_TROW - 1)],
                        emb_t.at[i], semt))
                for h in handles:
                    h.wait()

            h = load_xidx(0, cbase, 0)
            for j in range(_COLS):
                slot = j % 2
                if j + 1 < _COLS:
                    nh = load_xidx(j + 1, cbase, 1 - slot)
                h.wait()

                def pbody(g, pc):
                    voff = (xids[slot][pl.ds(g * _L, _L)] & 1) * _DIM
                    for l in range(_L):
                        i = g * _L + l
                        px = voff[l]
                        acc = None
                        for kk in range(_DIM // _L):
                            t = emb_t[i, pl.ds(kk * _L, _L)]
                            x = xbs[slot][i, pl.ds(px + kk * _L, _L)]
                            acc = t * x if acc is None else acc + t * x
                        part_v[pl.ds((i * _COLS + j) * _L, _L)] = acc
                    return pc
                lax.fori_loop(0, _CHUNK // _L, pbody, 0)
                if j + 1 < _COLS:
                    h = nh

            def ebody(i, ec):
                partials = [part_v[pl.ds((i * _COLS + j) * _L, _L)]
                            for j in range(_COLS)]
                dots_v[pl.ds(i * _COLS, _COLS)] = _transpose_sum(partials) * sign
                return ec
            lax.fori_loop(0, _CHUNK, ebody, 0)

            pltpu.sync_copy(dots_v,
                            out_hbm.at[pl.ds(cbase * _COLS, _CHUNK * _COLS)])
            return carry
        lax.fori_loop(0, _NCHUNK, cbody, 0)

    return k(target, context, neg, w3t, wlin)


def _tc_loss_body(dots_ref, out_ref):
    x = dots_ref[...]
    ls = jnp.minimum(x, 0.0) - jnp.log(1.0 + jnp.exp(-jnp.abs(x)))
    out_ref[...] = jnp.broadcast_to(-jnp.sum(ls) / _BATCH, (1, 1))


_tc_loss = pl.pallas_call(
    _tc_loss_body,
    out_shape=jax.ShapeDtypeStruct((1, 1), jnp.float32),
)


def kernel(target, context, W_target, W_context):
    neg = _neg_indices()
    w3t = W_target.reshape(_NTILE, _TROW, _DIM)
    w3c = W_context.reshape(_NTILE, _TROW, _DIM)
    wlin = _sc_compact(w3c)
    dots = _sc_dots(target.astype(jnp.int32), context.astype(jnp.int32),
                    neg, w3t, wlin)
    loss = _tc_loss(dots.reshape(_BATCH * _COLS // 128, 128))
    return loss[0, 0]


# speedup vs baseline: 1.0437x; 1.0061x over previous
import functools

import jax
import jax.numpy as jnp
from jax import lax
from jax.experimental import pallas as pl
from jax.experimental.pallas import tpu as pltpu
from jax.experimental.pallas import tpu_sc as plsc

_VOCAB = 1_000_000
_DIM = 64
_BATCH = 16384
_NEGS = 15
_COLS = _NEGS + 1

_NC = 2
_NS = 16
_NW = _NC * _NS
_L = 16
_BPW = _BATCH // _NW
_CHUNK = 32
_NCHUNK = _BPW // _CHUNK

_RBLK = 128
_RPW = 31232
_NBLK = -(-(_VOCAB - 31 * _RPW) // _RBLK)


def _neg_indices():
    neg = jax.random.randint(jax.random.key(12345), (_BATCH, _NEGS), 0, _VOCAB)
    return jnp.transpose(neg.astype(jnp.int32)).reshape(-1)


def _vperm(v, idx):
    return lax.gather(
        v, idx.reshape(_L, 1),
        dimension_numbers=lax.GatherDimensionNumbers(
            offset_dims=(), collapsed_slice_dims=(0,), start_index_map=(0,)),
        slice_sizes=(1,),
        mode=lax.GatherScatterMode.PROMISE_IN_BOUNDS)


def _transpose_sum(partials):
    lane = lax.iota(jnp.int32, _L)
    vecs = list(partials)
    for k in range(4):
        b = 1 << k
        lo_mask = ((lane >> k) & 1) == 0
        idx_dn = (lane - b) & (_L - 1)
        idx_up = (lane + b) & (_L - 1)
        new = [None] * _L
        for i in range(_L):
            if (i >> k) & 1:
                continue
            a, bb = vecs[i], vecs[i + b]
            new[i] = jnp.where(lo_mask, a, _vperm(bb, idx_dn))
            new[i + b] = jnp.where(lo_mask, _vperm(a, idx_up), bb)
        vecs = new
    while len(vecs) > 1:
        vecs = [vecs[i] + vecs[i + 1] for i in range(0, len(vecs), 2)]
    return vecs[0]


def _sc_compact(W):
    mesh = plsc.VectorSubcoreMesh(core_axis_name="c", subcore_axis_name="s")

    @functools.partial(
        pl.kernel,
        mesh=mesh,
        out_type=jax.ShapeDtypeStruct((_VOCAB // 2, 2 * _DIM), jnp.float32),
        scratch_types=[
            pltpu.VMEM((_RBLK, _DIM), jnp.float32),
            pltpu.VMEM((_RBLK, _DIM), jnp.float32),
            pltpu.VMEM((_RBLK // 2, 2 * _DIM), jnp.float32),
            pltpu.SemaphoreType.DMA,
            pltpu.SemaphoreType.DMA,
        ],
    )
    def k(w_hbm, out_hbm, buf0, buf1, packed, sem0, sem1):
        wid = lax.axis_index("s") * _NC + lax.axis_index("c")
        wstart = wid * _RPW
        bufs = (buf0, buf1)
        sems = (sem0, sem1)

        def rstart_of(b):
            return pl.multiple_of(
                jnp.minimum(wstart + b * _RBLK, _VOCAB - _RBLK), _RBLK)

        def compact_block(cur, b):
            for r in range(_RBLK):
                for kk in range(_DIM // _L):
                    packed[r // 2, pl.ds((r % 2) * _DIM + kk * _L, _L)] = (
                        cur[r, pl.ds(kk * _L, _L)])
            pltpu.sync_copy(
                packed,
                out_hbm.at[pl.ds(pl.multiple_of(rstart_of(b) // 2, _RBLK // 2),
                                 _RBLK // 2), :])

        pltpu.async_copy(w_hbm.at[pl.ds(rstart_of(0), _RBLK), :],
                         buf0, sem0).wait()

        def pbody(p, carry):
            for parity in range(2):
                b = p * 2 + parity
                nh = pltpu.async_copy(
                    w_hbm.at[pl.ds(rstart_of(b + 1), _RBLK), :],
                    bufs[1 - parity], sems[1 - parity])
                compact_block(bufs[parity], b)
                nh.wait()
            return carry
        lax.fori_loop(0, _NBLK // 2, pbody, 0)
        if _NBLK % 2 == 1:
            compact_block(bufs[0], _NBLK - 1)

    return k(W)


def _sc_dots(target, context, neg, W_target, wlin):
    mesh = plsc.VectorSubcoreMesh(core_axis_name="c", subcore_axis_name="s")

    @functools.partial(
        pl.kernel,
        mesh=mesh,
        out_type=jax.ShapeDtypeStruct((_BATCH * _COLS,), jnp.float32),
        scratch_types=[
            pltpu.VMEM((_CHUNK,), jnp.int32),
            pltpu.VMEM((_CHUNK,), jnp.int32),
            pltpu.VMEM((_CHUNK,), jnp.int32),
            pltpu.VMEM((_CHUNK,), jnp.int32),
            pltpu.VMEM((_CHUNK,), jnp.int32),
            pltpu.VMEM((_CHUNK, 8, _DIM), jnp.float32),
            pltpu.VMEM((_CHUNK, _DIM), jnp.float32),
            pltpu.VMEM((_CHUNK, 2 * _DIM), jnp.float32),
            pltpu.VMEM((_CHUNK, 2 * _DIM), jnp.float32),
            pltpu.VMEM((_CHUNK * _COLS * _L,), jnp.float32),
            pltpu.VMEM((_CHUNK * _COLS,), jnp.float32),
            pltpu.SemaphoreType.DMA,
            pltpu.SemaphoreType.DMA,
            pltpu.SemaphoreType.DMA,
        ],
    )
    def k(target_hbm, context_hbm, neg_hbm, wt_hbm, wlin_hbm, out_hbm,
          xid0, xid1, pid0, pid1, tid_v, emb_t8, emb_t, xb0, xb1, part_v,
          dots_v, sem0, sem1, semt):
        wid = lax.axis_index("s") * _NC + lax.axis_index("c")
        base = wid * _BPW
        lane = lax.iota(jnp.int32, _L)
        sign = jnp.where(lane == 0, 1.0, -1.0)
        xids = (xid0, xid1)
        pids = (pid0, pid1)
        xbs = (xb0, xb1)
        sems = (sem0, sem1)

        def load_xidx(j, cbase, slot):
            if j == 0:
                pltpu.sync_copy(context_hbm.at[pl.ds(cbase, _CHUNK)],
                                xids[slot])
            else:
                pltpu.sync_copy(
                    neg_hbm.at[pl.ds((j - 1) * _BATCH + cbase, _CHUNK)],
                    xids[slot])

            def qbody(g, carry):
                v = xids[slot][pl.ds(g * _L, _L)]
                pids[slot][pl.ds(g * _L, _L)] = lax.shift_right_logical(v, 1)
                return carry
            lax.fori_loop(0, _CHUNK // _L, qbody, 0)
            return pltpu.async_copy(wlin_hbm.at[pids[slot]], xbs[slot],
                                    sems[slot])

        def cbody(cb, carry):
            cbase = base + cb * _CHUNK
            pltpu.sync_copy(target_hbm.at[pl.ds(cbase, _CHUNK)], tid_v)
            for g in range(_CHUNK // _L):
                vt = tid_v[pl.ds(g * _L, _L)]
                handles = []
                for l in range(_L):
                    i = g * _L + l
                    r8 = pl.multiple_of((vt[l] >> 3) * 8, 8)
                    handles.append(pltpu.async_copy(
                        wt_hbm.at[pl.ds(r8, 8), :], emb_t8.at[i], semt))
                for h in handles:
                    h.wait()

            def tcompact(g, tc):
                vt = tid_v[pl.ds(g * _L, _L)] & 7
                for l in range(_L):
                    i = g * _L + l
                    for kk in range(_DIM // _L):
                        emb_t[i, pl.ds(kk * _L, _L)] = (
                            emb_t8[i, vt[l], pl.ds(kk * _L, _L)])
                return tc
            lax.fori_loop(0, _CHUNK // _L, tcompact, 0)

            h = load_xidx(0, cbase, 0)
            for j in range(_COLS):
                slot = j % 2
                if j + 1 < _COLS:
                    nh = load_xidx(j + 1, cbase, 1 - slot)
                h.wait()

                def pbody(g, pc):
                    vpar = xids[slot][pl.ds(g * _L, _L)] & 1
                    for l in range(_L):
                        i = g * _L + l
                        alo = None
                        ahi = None
                        for kk in range(_DIM // _L):
                            t = emb_t[i, pl.ds(kk * _L, _L)]
                            xl = xbs[slot][i, pl.ds(kk * _L, _L)]
                            xh = xbs[slot][i, pl.ds(_DIM + kk * _L, _L)]
                            alo = t * xl if alo is None else alo + t * xl
                            ahi = t * xh if ahi is None else ahi + t * xh
                        part_v[pl.ds((i * _COLS + j) * _L, _L)] = jnp.where(
                            vpar[l] != 0, ahi, alo)
                    return pc
                lax.fori_loop(0, _CHUNK // _L, pbody, 0)
                if j + 1 < _COLS:
                    h = nh

            def ebody(i, ec):
                partials = [part_v[pl.ds((i * _COLS + j) * _L, _L)]
                            for j in range(_COLS)]
                dots_v[pl.ds(i * _COLS, _COLS)] = _transpose_sum(partials) * sign
                return ec
            lax.fori_loop(0, _CHUNK, ebody, 0)

            pltpu.sync_copy(dots_v,
                            out_hbm.at[pl.ds(cbase * _COLS, _CHUNK * _COLS)])
            return carry
        lax.fori_loop(0, _NCHUNK, cbody, 0)

    return k(target, context, neg, W_target, wlin)


def _tc_loss_body(dots_ref, out_ref):
    x = dots_ref[...]
    ls = jnp.minimum(x, 0.0) - jnp.log(1.0 + jnp.exp(-jnp.abs(x)))
    out_ref[...] = jnp.broadcast_to(-jnp.sum(ls) / _BATCH, (1, 1))


_tc_loss = pl.pallas_call(
    _tc_loss_body,
    out_shape=jax.ShapeDtypeStruct((1, 1), jnp.float32),
)


def kernel(target, context, W_target, W_context):
    neg = _neg_indices()
    wlin = _sc_compact(W_context)
    dots = _sc_dots(target.astype(jnp.int32), context.astype(jnp.int32),
                    neg, W_target, wlin)
    loss = _tc_loss(dots.reshape(_BATCH * _COLS // 128, 128))
    return loss[0, 0]
